# Initial kernel scaffold; baseline (speedup 1.0000x reference)
import jax
import jax.numpy as jnp
from jax.experimental import pallas as pl


def kernel(x, edge_index, mask, gat_W, gat_b, gat_a_src, gat_a_dst, Wq, bq, Wk, bk, Wv, bv, Wo, bo, W1, b1, W2, b2, W3, b3):
    raise NotImplementedError("write your pallas kernel here")


# speedup vs baseline: 1.0014x; 1.0014x over previous
import functools

import jax
import jax.numpy as jnp
from jax.experimental import pallas as pl
from jax.experimental.pallas import tpu as pltpu

N = 4096
D = 3
L = 10
QT = 256


def _attn_mlp_body(h_ref, hT_ref, mask_ref,
                   Wq_ref, bq_ref, WkT_ref, bk_ref, WvT_ref, bv_ref,
                   Wo_ref, bo_ref, W1_ref, b1_ref, W2_ref, b2_ref,
                   W3_ref, b3_ref, out_ref):
    h_t = h_ref[...]
    hT = hT_ref[...]
    Q = jnp.dot(h_t, Wq_ref[...], preferred_element_type=jnp.float32) + bq_ref[...][None, :]
    kT = jnp.dot(WkT_ref[...], hT, preferred_element_type=jnp.float32) + bk_ref[...][:, None]
    vT = jnp.dot(WvT_ref[...], hT, preferred_element_type=jnp.float32) + bv_ref[...][:, None]
    cols = []
    for hh in range(3):
        qcol = Q[:, hh:hh + 1]
        krow = kT[hh:hh + 1, :]
        vrow = vT[hh:hh + 1, :]
        kmax = jnp.max(krow, axis=1, keepdims=True)
        kmin = jnp.min(krow, axis=1, keepdims=True)
        m = jnp.where(qcol >= 0.0, qcol * kmax, qcol * kmin)
        ex = jnp.exp(qcol * krow - m)
        den = jnp.sum(ex, axis=1, keepdims=True)
        num = jnp.sum(ex * vrow, axis=1, keepdims=True)
        cols.append(num / den)
    mo = jnp.concatenate(cols, axis=1)
    mo = jnp.dot(mo, Wo_ref[...], preferred_element_type=jnp.float32) + bo_ref[...][None, :]
    hm = jnp.maximum(jnp.dot(mo, W1_ref[...], preferred_element_type=jnp.float32) + b1_ref[...][None, :], 0.0)
    hm = jnp.maximum(jnp.dot(hm, W2_ref[...], preferred_element_type=jnp.float32) + b2_ref[...][None, :], 0.0)
    logits = jnp.dot(hm, W3_ref[...], preferred_element_type=jnp.float32) + b3_ref[...][None, :]
    out_ref[...] = jnp.where(mask_ref[...] == 1, logits, 0.0)


def _attn_mlp(h, hT, mask2d, Wq, bq, WkT, bk, WvT, bv, Wo, bo, W13, b1, W2, b2, W3, b3):
    full = lambda shape: pl.BlockSpec(shape, lambda i: tuple(0 for _ in shape))
    grid = N // QT
    return pl.pallas_call(
        _attn_mlp_body,
        grid=(grid,),
        in_specs=[
            pl.BlockSpec((QT, D), lambda i: (i, 0)),
            full((D, N)),
            pl.BlockSpec((QT, 1), lambda i: (i, 0)),
            full((D, D)), full((D,)), full((D, D)), full((D,)),
            full((D, D)), full((D,)), full((D, D)), full((D,)),
            full((D, 16)), full((16,)), full((16, 32)), full((32,)),
            full((32, 1)), full((1,)),
        ],
        out_specs=pl.BlockSpec((QT, 1), lambda i: (i, 0)),
        out_shape=jax.ShapeDtypeStruct((N, 1), jnp.float32),
    )(h, hT, mask2d, Wq, bq, WkT, bk, WvT, bv, Wo, bo, W13, b1, W2, b2, W3, b3)


def kernel(x, edge_index, mask, gat_W, gat_b, gat_a_src, gat_a_dst, Wq, bq, Wk, bk, Wv, bv, Wo, bo, W1, b1, W2, b2, W3, b3):
    n = x.shape[0]
    loops = jnp.arange(n, dtype=edge_index.dtype)
    ei = jnp.concatenate([edge_index, jnp.stack([loops, loops])], axis=1)
    src, dst = ei[0], ei[1]
    h = x
    for l in range(L):
        hw = h @ gat_W[l]
        logit = jax.nn.leaky_relu((hw * gat_a_src[l]).sum(-1)[src] + (hw * gat_a_dst[l]).sum(-1)[dst], 0.2)
        m = jax.ops.segment_max(logit, dst, num_segments=n)
        ex = jnp.exp(logit - m[dst])
        den = jax.ops.segment_sum(ex, dst, num_segments=n)
        alpha = ex / (den[dst] + 1e-16)
        h_new = jax.ops.segment_sum(alpha[:, None] * hw[src], dst, num_segments=n) + gat_b[l]
        h = jax.nn.relu(h_new) if l < L - 1 else h_new

    logits = _attn_mlp(h, h.T, mask[:, None].astype(jnp.int32),
                       Wq, bq, Wk.T, bk, Wv.T, bv, Wo, bo,
                       W1[:3, :], b1, W2, b2, W3, b3)
    return logits[:, 0]


# speedup vs baseline: 33.2865x; 33.2395x over previous
import functools

import jax
import jax.numpy as jnp
from jax import lax
from jax.experimental import pallas as pl
from jax.experimental.pallas import tpu as pltpu
from jax.experimental.pallas import tpu_sc as plsc

NN = 4096
DD = 3
NL = 10
QT = 256
EB = 8192
NSUB = 16
OWN = NN // NSUB
NEG = float("-inf")


def _gat_sc_body(src_hbm, dst_hbm, off_hbm, xT_hbm, wv_hbm, hT_out,
                 eb_src, eb_dst, offs_v, wbuf,
                 h0, h1, h2, hw0, hw1, hw2, ssrc, sdst, m, den,
                 ac0, ac1, ac2):
    wid = lax.axis_index("s")
    base = wid * OWN
    lane = lax.iota(jnp.int32, 16)

    pltpu.sync_copy(off_hbm, offs_v)
    pltpu.sync_copy(wv_hbm, wbuf)
    pltpu.sync_copy(xT_hbm.at[pl.ds(0, NN)], h0)
    pltpu.sync_copy(xT_hbm.at[pl.ds(NN, NN)], h1)
    pltpu.sync_copy(xT_hbm.at[pl.ds(2 * NN, NN)], h2)

    def lane_sel(vec, i):
        return jnp.max(jnp.where(lane == i, vec, jnp.int32(-2147483648)))

    oc0 = offs_v[pl.ds(0, 16)]
    oc1 = offs_v[pl.ds(16, 16)]
    start = lane_sel(oc0, wid)
    end = jnp.where(wid == NSUB - 1, lane_sel(oc1, 0), lane_sel(oc0, wid + 1))
    start8 = (start // 8) * 8
    nblocks = (end - start8 + (EB - 1)) // EB

    def wb(i):
        return plsc.load_gather(wbuf, [jnp.zeros((16,), jnp.int32) + i])

    def layer_body(l, carry):
        w = [wb(9 * l + t) for t in range(9)]
        asv = [wb(90 + 3 * l + t) for t in range(3)]
        adv = [wb(120 + 3 * l + t) for t in range(3)]
        bv = [wb(150 + 3 * l + t) for t in range(3)]

        def dense_chunk(i, _):
            sl = pl.ds(i * 16, 16)
            a, b_, c = h0[sl], h1[sl], h2[sl]
            t0 = a * w[0] + b_ * w[3] + c * w[6]
            t1 = a * w[1] + b_ * w[4] + c * w[7]
            t2 = a * w[2] + b_ * w[5] + c * w[8]
            hw0[sl] = t0
            hw1[sl] = t1
            hw2[sl] = t2
            ssrc[sl] = t0 * asv[0] + t1 * asv[1] + t2 * asv[2]
            sdst[sl] = t0 * adv[0] + t1 * adv[1] + t2 * adv[2]
            return 0
        lax.fori_loop(0, NN // 16, dense_chunk, 0)

        def init_chunk(i, _):
            sl = pl.ds(base + i * 16, 16)
            m[sl] = jnp.full((16,), NEG, jnp.float32)
            den[sl] = jnp.zeros((16,), jnp.float32)
            ac0[sl] = jnp.zeros((16,), jnp.float32)
            ac1[sl] = jnp.zeros((16,), jnp.float32)
            ac2[sl] = jnp.zeros((16,), jnp.float32)
            return 0
        lax.fori_loop(0, OWN // 16, init_chunk, 0)

        def sweep(chunk_fn):
            def blk(b, _):
                bstart = start8 + b * EB
                pltpu.sync_copy(src_hbm.at[pl.ds(bstart, EB)], eb_src)
                pltpu.sync_copy(dst_hbm.at[pl.ds(bstart, EB)], eb_dst)
                nch = jnp.minimum((end - bstart + 15) // 16, EB // 16)

                def ch(ci, _):
                    sl = pl.ds(ci * 16, 16)
                    sv = eb_src[sl]
                    dv = eb_dst[sl]
                    eid = bstart + ci * 16 + lane
                    valid = (eid >= start) & (eid < end)
                    chunk_fn(sv, dv, valid)
                    return 0
                lax.fori_loop(0, nch, ch, 0)
                return 0
            lax.fori_loop(0, nblocks, blk, 0)

        def logit_of(sv, dv):
            lg = plsc.load_gather(ssrc, [sv]) + plsc.load_gather(sdst, [dv])
            return jnp.maximum(lg, 0.2 * lg)

        def s1(sv, dv, valid):
            logit = logit_of(sv, dv)

            def cond(p):
                return jnp.max(p) > 0

            def body(p):
                cur = plsc.load_gather(m, [dv])
                need = (p > 0) & (logit > cur)
                plsc.store_scatter(m, [dv], logit, mask=need)
                cur2 = plsc.load_gather(m, [dv])
                return jnp.where(need & (cur2 < logit), 1, 0).astype(jnp.int32)

            lax.while_loop(cond, body, valid.astype(jnp.int32))
        sweep(s1)

        def s2(sv, dv, valid):
            logit = logit_of(sv, dv)
            mg = plsc.load_gather(m, [dv])
            ex = jnp.exp(logit - mg)
            ex = jnp.where(valid, ex, 0.0)
            plsc.addupdate_scatter(den, [dv], ex, mask=valid)
            g0 = plsc.load_gather(hw0, [sv])
            g1 = plsc.load_gather(hw1, [sv])
            g2 = plsc.load_gather(hw2, [sv])
            plsc.addupdate_scatter(ac0, [dv], ex * g0, mask=valid)
            plsc.addupdate_scatter(ac1, [dv], ex * g1, mask=valid)
            plsc.addupdate_scatter(ac2, [dv], ex * g2, mask=valid)
        sweep(s2)

        def upd_chunk(i, _):
            sl = pl.ds(base + i * 16, 16)
            d_ = den[sl] + 1e-16
            n0 = ac0[sl] / d_ + bv[0]
            n1 = ac1[sl] / d_ + bv[1]
            n2 = ac2[sl] / d_ + bv[2]
            last = l >= NL - 1
            h0[sl] = jnp.where(last, n0, jnp.maximum(n0, 0.0))
            h1[sl] = jnp.where(last, n1, jnp.maximum(n1, 0.0))
            h2[sl] = jnp.where(last, n2, jnp.maximum(n2, 0.0))
            return 0
        lax.fori_loop(0, OWN // 16, upd_chunk, 0)

        pltpu.sync_copy(h0.at[pl.ds(base, OWN)], hT_out.at[pl.ds(base, OWN)])
        pltpu.sync_copy(h1.at[pl.ds(base, OWN)], hT_out.at[pl.ds(NN + base, OWN)])
        pltpu.sync_copy(h2.at[pl.ds(base, OWN)], hT_out.at[pl.ds(2 * NN + base, OWN)])
        plsc.subcore_barrier()
        pltpu.sync_copy(hT_out.at[pl.ds(0, NN)], h0)
        pltpu.sync_copy(hT_out.at[pl.ds(NN, NN)], h1)
        pltpu.sync_copy(hT_out.at[pl.ds(2 * NN, NN)], h2)
        plsc.subcore_barrier()
        return carry

    lax.fori_loop(0, NL, layer_body, 0)


def _gat_sc(src_pad, dst_pad, off, xT, wvec):
    mesh = plsc.VectorSubcoreMesh(core_axis_name="c", subcore_axis_name="s",
                                  num_cores=1, num_subcores=NSUB)
    fvec = lambda n: pltpu.VMEM((n,), jnp.float32)
    return pl.kernel(
        _gat_sc_body,
        out_type=jax.ShapeDtypeStruct((DD * NN,), jnp.float32),
        mesh=mesh,
        scratch_types=[
            pltpu.VMEM((EB,), jnp.int32),
            pltpu.VMEM((EB,), jnp.int32),
            pltpu.VMEM((32,), jnp.int32),
            fvec(192),
            fvec(NN), fvec(NN), fvec(NN),
            fvec(NN), fvec(NN), fvec(NN),
            fvec(NN), fvec(NN),
            fvec(NN), fvec(NN),
            fvec(NN), fvec(NN), fvec(NN),
        ],
        compiler_params=pltpu.CompilerParams(needs_layout_passes=False),
    )(src_pad, dst_pad, off, xT, wvec)


def _attn_mlp_body(h_ref, hT_ref, mask_ref,
                   Wq_ref, bq_ref, WkT_ref, bk_ref, WvT_ref, bv_ref,
                   Wo_ref, bo_ref, W1_ref, b1_ref, W2_ref, b2_ref,
                   W3_ref, b3_ref, out_ref):
    h_t = h_ref[...]
    hT = hT_ref[...]
    Q = jnp.dot(h_t, Wq_ref[...], preferred_element_type=jnp.float32) + bq_ref[...][None, :]
    kT = jnp.dot(WkT_ref[...], hT, preferred_element_type=jnp.float32) + bk_ref[...][:, None]
    vT = jnp.dot(WvT_ref[...], hT, preferred_element_type=jnp.float32) + bv_ref[...][:, None]
    cols = []
    for hh in range(3):
        qcol = Q[:, hh:hh + 1]
        krow = kT[hh:hh + 1, :]
        vrow = vT[hh:hh + 1, :]
        kmax = jnp.max(krow, axis=1, keepdims=True)
        kmin = jnp.min(krow, axis=1, keepdims=True)
        mrow = jnp.where(qcol >= 0.0, qcol * kmax, qcol * kmin)
        ex = jnp.exp(qcol * krow - mrow)
        dn = jnp.sum(ex, axis=1, keepdims=True)
        nm = jnp.sum(ex * vrow, axis=1, keepdims=True)
        cols.append(nm / dn)
    mo = jnp.concatenate(cols, axis=1)
    mo = jnp.dot(mo, Wo_ref[...], preferred_element_type=jnp.float32) + bo_ref[...][None, :]
    hm = jnp.maximum(jnp.dot(mo, W1_ref[...], preferred_element_type=jnp.float32) + b1_ref[...][None, :], 0.0)
    hm = jnp.maximum(jnp.dot(hm, W2_ref[...], preferred_element_type=jnp.float32) + b2_ref[...][None, :], 0.0)
    logits = jnp.dot(hm, W3_ref[...], preferred_element_type=jnp.float32) + b3_ref[...][None, :]
    out_ref[...] = jnp.where(mask_ref[...] == 1, logits, 0.0)


def _attn_mlp(h, hT, mask2d, Wq, bq, WkT, bk, WvT, bv, Wo, bo, W13, b1, W2, b2, W3, b3):
    full = lambda shape: pl.BlockSpec(shape, lambda i: tuple(0 for _ in shape))
    grid = NN // QT
    return pl.pallas_call(
        _attn_mlp_body,
        grid=(grid,),
        in_specs=[
            pl.BlockSpec((QT, DD), lambda i: (i, 0)),
            full((DD, NN)),
            pl.BlockSpec((QT, 1), lambda i: (i, 0)),
            full((DD, DD)), full((DD,)), full((DD, DD)), full((DD,)),
            full((DD, DD)), full((DD,)), full((DD, DD)), full((DD,)),
            full((DD, 16)), full((16,)), full((16, 32)), full((32,)),
            full((32, 1)), full((1,)),
        ],
        out_specs=pl.BlockSpec((QT, 1), lambda i: (i, 0)),
        out_shape=jax.ShapeDtypeStruct((NN, 1), jnp.float32),
    )(h, hT, mask2d, Wq, bq, WkT, bk, WvT, bv, Wo, bo, W13, b1, W2, b2, W3, b3)


def kernel(x, edge_index, mask, gat_W, gat_b, gat_a_src, gat_a_dst, Wq, bq, Wk, bk, Wv, bv, Wo, bo, W1, b1, W2, b2, W3, b3):
    n = x.shape[0]
    loops = jnp.arange(n, dtype=edge_index.dtype)
    src_all = jnp.concatenate([edge_index[0], loops])
    dst_all = jnp.concatenate([edge_index[1], loops])
    order = jnp.argsort(dst_all)
    src_s = jnp.concatenate([src_all[order], jnp.zeros((EB,), jnp.int32)])
    dst_s = jnp.concatenate([dst_all[order], jnp.zeros((EB,), jnp.int32)])
    bnds = jnp.arange(0, n + 1, OWN, dtype=jnp.int32)
    off = jnp.searchsorted(dst_all[order], bnds).astype(jnp.int32)
    off = jnp.concatenate([off, jnp.zeros((32 - off.shape[0],), jnp.int32)])

    wvec = jnp.concatenate([
        gat_W.reshape(-1), gat_a_src.reshape(-1), gat_a_dst.reshape(-1),
        gat_b.reshape(-1), jnp.zeros((12,), jnp.float32)])

    hT = _gat_sc(src_s, dst_s, off, x.T.reshape(-1), wvec).reshape(DD, NN)
    h = hT.T

    logits = _attn_mlp(h, hT, mask[:, None].astype(jnp.int32),
                       Wq, bq, Wk.T, bk, Wv.T, bv, Wo, bo,
                       W1[:3, :], b1, W2, b2, W3, b3)
    return logits[:, 0]


# speedup vs baseline: 43.5307x; 1.3078x over previous
import functools

import jax
import jax.numpy as jnp
from jax import lax
from jax.experimental import pallas as pl
from jax.experimental.pallas import tpu as pltpu
from jax.experimental.pallas import tpu_sc as plsc

NN = 4096
DD = 3
NL = 10
QT = 256
EB = 8192
NSUB = 16
OWN = NN // NSUB
NEG = float("-inf")


def _gat_sc_body(src_hbm, dst_hbm, off_hbm, xT_hbm, wv_hbm, hT_out,
                 eb_src, eb_dst, offs_v, wbuf,
                 h0, h1, h2, hw0, hw1, hw2, ssrc, sdst, m, den,
                 ac0, ac1, ac2):
    wid = lax.axis_index("s")
    base = wid * OWN
    lane = lax.iota(jnp.int32, 16)

    pltpu.sync_copy(off_hbm, offs_v)
    pltpu.sync_copy(wv_hbm, wbuf)
    pltpu.sync_copy(xT_hbm.at[pl.ds(0, NN)], h0)
    pltpu.sync_copy(xT_hbm.at[pl.ds(NN, NN)], h1)
    pltpu.sync_copy(xT_hbm.at[pl.ds(2 * NN, NN)], h2)

    def lane_sel(vec, i):
        return jnp.max(jnp.where(lane == i, vec, jnp.int32(-2147483648)))

    oc0 = offs_v[pl.ds(0, 16)]
    oc1 = offs_v[pl.ds(16, 16)]
    start = lane_sel(oc0, wid)
    end = jnp.where(wid == NSUB - 1, lane_sel(oc1, 0), lane_sel(oc0, wid + 1))
    start8 = (start // 8) * 8
    nblocks = (end - start8 + (EB - 1)) // EB

    def wb(i):
        return plsc.load_gather(wbuf, [jnp.zeros((16,), jnp.int32) + i])

    def layer_body(l, carry):
        w = [wb(9 * l + t) for t in range(9)]
        asv = [wb(90 + 3 * l + t) for t in range(3)]
        adv = [wb(120 + 3 * l + t) for t in range(3)]
        bv = [wb(150 + 3 * l + t) for t in range(3)]

        def dense_chunk(i, _):
            sl = pl.ds(i * 16, 16)
            a, b_, c = h0[sl], h1[sl], h2[sl]
            t0 = a * w[0] + b_ * w[3] + c * w[6]
            t1 = a * w[1] + b_ * w[4] + c * w[7]
            t2 = a * w[2] + b_ * w[5] + c * w[8]
            hw0[sl] = t0
            hw1[sl] = t1
            hw2[sl] = t2
            ssrc[sl] = t0 * asv[0] + t1 * asv[1] + t2 * asv[2]
            sdst[sl] = t0 * adv[0] + t1 * adv[1] + t2 * adv[2]
            return 0
        lax.fori_loop(0, NN // 16, dense_chunk, 0)

        def init_chunk(i, _):
            sl = pl.ds(base + i * 16, 16)
            m[sl] = jnp.full((16,), NEG, jnp.float32)
            den[sl] = jnp.zeros((16,), jnp.float32)
            ac0[sl] = jnp.zeros((16,), jnp.float32)
            ac1[sl] = jnp.zeros((16,), jnp.float32)
            ac2[sl] = jnp.zeros((16,), jnp.float32)
            return 0
        lax.fori_loop(0, OWN // 16, init_chunk, 0)

        def sweep(chunk_fn):
            def blk(b, _):
                bstart = start8 + b * EB
                pltpu.sync_copy(src_hbm.at[pl.ds(bstart, EB)], eb_src)
                pltpu.sync_copy(dst_hbm.at[pl.ds(bstart, EB)], eb_dst)
                nch = jnp.minimum((end - bstart + 15) // 16, EB // 16)

                def ch(ci, _):
                    sl = pl.ds(ci * 16, 16)
                    sv = eb_src[sl]
                    dv = eb_dst[sl]
                    eid = bstart + ci * 16 + lane
                    valid = (eid >= start) & (eid < end)
                    chunk_fn(sv, dv, valid)
                    return 0
                lax.fori_loop(0, nch, ch, 0)
                return 0
            lax.fori_loop(0, nblocks, blk, 0)

        def logit_of(sv, dv):
            lg = plsc.load_gather(ssrc, [sv]) + plsc.load_gather(sdst, [dv])
            return jnp.maximum(lg, 0.2 * lg)

        def s1(sv, dv, valid):
            logit = logit_of(sv, dv)

            def cond(p):
                return jnp.max(p) > 0

            def body(p):
                cur = plsc.load_gather(m, [dv])
                need = (p > 0) & (logit > cur)
                plsc.store_scatter(m, [dv], logit, mask=need)
                cur2 = plsc.load_gather(m, [dv])
                return jnp.where(need & (cur2 < logit), 1, 0).astype(jnp.int32)

            lax.while_loop(cond, body, valid.astype(jnp.int32))
        sweep(s1)

        def s2(sv, dv, valid):
            logit = logit_of(sv, dv)
            mg = plsc.load_gather(m, [dv])
            ex = jnp.exp(logit - mg)
            ex = jnp.where(valid, ex, 0.0)
            plsc.addupdate_scatter(den, [dv], ex, mask=valid)
            g0 = plsc.load_gather(hw0, [sv])
            g1 = plsc.load_gather(hw1, [sv])
            g2 = plsc.load_gather(hw2, [sv])
            plsc.addupdate_scatter(ac0, [dv], ex * g0, mask=valid)
            plsc.addupdate_scatter(ac1, [dv], ex * g1, mask=valid)
            plsc.addupdate_scatter(ac2, [dv], ex * g2, mask=valid)
        sweep(s2)

        def upd_chunk(i, _):
            sl = pl.ds(base + i * 16, 16)
            d_ = den[sl] + 1e-16
            n0 = ac0[sl] / d_ + bv[0]
            n1 = ac1[sl] / d_ + bv[1]
            n2 = ac2[sl] / d_ + bv[2]
            last = l >= NL - 1
            h0[sl] = jnp.where(last, n0, jnp.maximum(n0, 0.0))
            h1[sl] = jnp.where(last, n1, jnp.maximum(n1, 0.0))
            h2[sl] = jnp.where(last, n2, jnp.maximum(n2, 0.0))
            return 0
        lax.fori_loop(0, OWN // 16, upd_chunk, 0)

        pltpu.sync_copy(h0.at[pl.ds(base, OWN)], hT_out.at[pl.ds(base, OWN)])
        pltpu.sync_copy(h1.at[pl.ds(base, OWN)], hT_out.at[pl.ds(NN + base, OWN)])
        pltpu.sync_copy(h2.at[pl.ds(base, OWN)], hT_out.at[pl.ds(2 * NN + base, OWN)])
        plsc.subcore_barrier()
        pltpu.sync_copy(hT_out.at[pl.ds(0, NN)], h0)
        pltpu.sync_copy(hT_out.at[pl.ds(NN, NN)], h1)
        pltpu.sync_copy(hT_out.at[pl.ds(2 * NN, NN)], h2)
        plsc.subcore_barrier()
        return carry

    lax.fori_loop(0, NL, layer_body, 0)


def _gat_sc(src_pad, dst_pad, off, xT, wvec):
    mesh = plsc.VectorSubcoreMesh(core_axis_name="c", subcore_axis_name="s",
                                  num_cores=1, num_subcores=NSUB)
    fvec = lambda n: pltpu.VMEM((n,), jnp.float32)
    return pl.kernel(
        _gat_sc_body,
        out_type=jax.ShapeDtypeStruct((DD * NN,), jnp.float32),
        mesh=mesh,
        scratch_types=[
            pltpu.VMEM((EB,), jnp.int32),
            pltpu.VMEM((EB,), jnp.int32),
            pltpu.VMEM((32,), jnp.int32),
            fvec(192),
            fvec(NN), fvec(NN), fvec(NN),
            fvec(NN), fvec(NN), fvec(NN),
            fvec(NN), fvec(NN),
            fvec(NN), fvec(NN),
            fvec(NN), fvec(NN), fvec(NN),
        ],
        compiler_params=pltpu.CompilerParams(needs_layout_passes=False),
    )(src_pad, dst_pad, off, xT, wvec)


def _attn_mlp_body(h_ref, hT_ref, mask_ref,
                   Wq_ref, bq_ref, WkT_ref, bk_ref, WvT_ref, bv_ref,
                   Wo_ref, bo_ref, W1_ref, b1_ref, W2_ref, b2_ref,
                   W3_ref, b3_ref, out_ref):
    h_t = h_ref[...]
    hT = hT_ref[...]
    Q = jnp.dot(h_t, Wq_ref[...], preferred_element_type=jnp.float32) + bq_ref[...][None, :]
    kT = jnp.dot(WkT_ref[...], hT, preferred_element_type=jnp.float32) + bk_ref[...][:, None]
    vT = jnp.dot(WvT_ref[...], hT, preferred_element_type=jnp.float32) + bv_ref[...][:, None]
    cols = []
    for hh in range(3):
        qcol = Q[:, hh:hh + 1]
        krow = kT[hh:hh + 1, :]
        vrow = vT[hh:hh + 1, :]
        kmax = jnp.max(krow, axis=1, keepdims=True)
        kmin = jnp.min(krow, axis=1, keepdims=True)
        mrow = jnp.where(qcol >= 0.0, qcol * kmax, qcol * kmin)
        ex = jnp.exp(qcol * krow - mrow)
        dn = jnp.sum(ex, axis=1, keepdims=True)
        nm = jnp.sum(ex * vrow, axis=1, keepdims=True)
        cols.append(nm / dn)
    mo = jnp.concatenate(cols, axis=1)
    mo = jnp.dot(mo, Wo_ref[...], preferred_element_type=jnp.float32) + bo_ref[...][None, :]
    hm = jnp.maximum(jnp.dot(mo, W1_ref[...], preferred_element_type=jnp.float32) + b1_ref[...][None, :], 0.0)
    hm = jnp.maximum(jnp.dot(hm, W2_ref[...], preferred_element_type=jnp.float32) + b2_ref[...][None, :], 0.0)
    logits = jnp.dot(hm, W3_ref[...], preferred_element_type=jnp.float32) + b3_ref[...][None, :]
    out_ref[...] = jnp.where(mask_ref[...] == 1, logits, 0.0)


def _attn_mlp(h, hT, mask2d, Wq, bq, WkT, bk, WvT, bv, Wo, bo, W13, b1, W2, b2, W3, b3):
    full = lambda shape: pl.BlockSpec(shape, lambda i: tuple(0 for _ in shape))
    grid = NN // QT
    return pl.pallas_call(
        _attn_mlp_body,
        grid=(grid,),
        in_specs=[
            pl.BlockSpec((QT, DD), lambda i: (i, 0)),
            full((DD, NN)),
            pl.BlockSpec((QT, 1), lambda i: (i, 0)),
            full((DD, DD)), full((DD,)), full((DD, DD)), full((DD,)),
            full((DD, DD)), full((DD,)), full((DD, DD)), full((DD,)),
            full((DD, 16)), full((16,)), full((16, 32)), full((32,)),
            full((32, 1)), full((1,)),
        ],
        out_specs=pl.BlockSpec((QT, 1), lambda i: (i, 0)),
        out_shape=jax.ShapeDtypeStruct((NN, 1), jnp.float32),
    )(h, hT, mask2d, Wq, bq, WkT, bk, WvT, bv, Wo, bo, W13, b1, W2, b2, W3, b3)


def kernel(x, edge_index, mask, gat_W, gat_b, gat_a_src, gat_a_dst, Wq, bq, Wk, bk, Wv, bv, Wo, bo, W1, b1, W2, b2, W3, b3):
    n = x.shape[0]
    loops = jnp.arange(n, dtype=edge_index.dtype)
    src_all = jnp.concatenate([edge_index[0], loops])
    dst_all = jnp.concatenate([edge_index[1], loops])
    order = jnp.argsort(dst_all // OWN, stable=True)
    src_s = jnp.concatenate([src_all[order], jnp.zeros((EB,), jnp.int32)])
    dst_s = jnp.concatenate([dst_all[order], jnp.zeros((EB,), jnp.int32)])
    bnds = jnp.arange(0, NSUB + 1, dtype=jnp.int32)
    off = jnp.searchsorted((dst_all // OWN)[order], bnds).astype(jnp.int32)
    off = jnp.concatenate([off, jnp.zeros((32 - off.shape[0],), jnp.int32)])

    wvec = jnp.concatenate([
        gat_W.reshape(-1), gat_a_src.reshape(-1), gat_a_dst.reshape(-1),
        gat_b.reshape(-1), jnp.zeros((12,), jnp.float32)])

    hT = _gat_sc(src_s, dst_s, off, x.T.reshape(-1), wvec).reshape(DD, NN)
    h = hT.T

    logits = _attn_mlp(h, hT, mask[:, None].astype(jnp.int32),
                       Wq, bq, Wk.T, bk, Wv.T, bv, Wo, bo,
                       W1[:3, :], b1, W2, b2, W3, b3)
    return logits[:, 0]


# speedup vs baseline: 51.7153x; 1.1880x over previous
import functools

import jax
import jax.numpy as jnp
from jax import lax
from jax.experimental import pallas as pl
from jax.experimental.pallas import tpu as pltpu
from jax.experimental.pallas import tpu_sc as plsc

NN = 4096
DD = 3
NL = 10
QT = 256
EB = 8192
EE = 65536
CAP = 8192
NSUB = 16
OWN = NN // NSUB
NEG = float("-inf")


def _gat_sc_body(src_hbm, dst_hbm, xT_hbm, wv_hbm, hT_out,
                 eb_src, eb_dst, Lsrc, Ldst, wbuf,
                 h0, h1, h2, hw0, hw1, hw2, ssrc, sdst, m, den,
                 ac0, ac1, ac2):
    wid = lax.axis_index("s")
    base = wid * OWN
    lane = lax.iota(jnp.int32, 16)

    pltpu.sync_copy(wv_hbm, wbuf)
    pltpu.sync_copy(xT_hbm.at[pl.ds(0, NN)], h0)
    pltpu.sync_copy(xT_hbm.at[pl.ds(NN, NN)], h1)
    pltpu.sync_copy(xT_hbm.at[pl.ds(2 * NN, NN)], h2)

    def zero_chunk(i, _):
        sl = pl.ds(i * 16, 16)
        z = jnp.zeros((16,), jnp.int32)
        Lsrc[sl] = z
        Ldst[sl] = z
        return 0
    lax.fori_loop(0, CAP // 16, zero_chunk, 0)

    def loop_chunk(i, _):
        sl = pl.ds(i * 16, 16)
        ids = base + i * 16 + lane
        Lsrc[sl] = ids
        Ldst[sl] = ids
        return 0
    lax.fori_loop(0, OWN // 16, loop_chunk, 0)

    def scan_blk(b, countv):
        pltpu.sync_copy(src_hbm.at[pl.ds(b * EB, EB)], eb_src)
        pltpu.sync_copy(dst_hbm.at[pl.ds(b * EB, EB)], eb_dst)

        def ch(ci, cv):
            sl = pl.ds(ci * 16, 16)
            sv = eb_src[sl]
            dv = eb_dst[sl]
            mk = (dv // OWN) == wid
            cs = plsc.cumsum(mk.astype(jnp.int32))
            pos = cv + cs - 1
            ok = mk & (pos < CAP)
            plsc.store_scatter(Lsrc, [pos], sv, mask=ok)
            plsc.store_scatter(Ldst, [pos], dv, mask=ok)
            return cv + plsc.all_reduce_population_count(mk)
        return lax.fori_loop(0, EB // 16, ch, countv)

    countv = lax.fori_loop(0, EE // EB, scan_blk,
                           jnp.zeros((16,), jnp.int32) + OWN)
    count_true = jnp.max(countv)
    count_eff = jnp.minimum(count_true, CAP)
    nch_list = (count_eff + 15) // 16

    def wb(i):
        return plsc.load_gather(wbuf, [jnp.zeros((16,), jnp.int32) + i])

    def layer_body(l, carry):
        w = [wb(9 * l + t) for t in range(9)]
        asv = [wb(90 + 3 * l + t) for t in range(3)]
        adv = [wb(120 + 3 * l + t) for t in range(3)]
        bv = [wb(150 + 3 * l + t) for t in range(3)]

        def dense_chunk(i, _):
            sl = pl.ds(i * 16, 16)
            a, b_, c = h0[sl], h1[sl], h2[sl]
            t0 = a * w[0] + b_ * w[3] + c * w[6]
            t1 = a * w[1] + b_ * w[4] + c * w[7]
            t2 = a * w[2] + b_ * w[5] + c * w[8]
            hw0[sl] = t0
            hw1[sl] = t1
            hw2[sl] = t2
            ssrc[sl] = t0 * asv[0] + t1 * asv[1] + t2 * asv[2]
            sdst[sl] = t0 * adv[0] + t1 * adv[1] + t2 * adv[2]
            return 0
        lax.fori_loop(0, NN // 16, dense_chunk, 0)

        def init_chunk(i, _):
            sl = pl.ds(base + i * 16, 16)
            m[sl] = jnp.full((16,), NEG, jnp.float32)
            den[sl] = jnp.zeros((16,), jnp.float32)
            ac0[sl] = jnp.zeros((16,), jnp.float32)
            ac1[sl] = jnp.zeros((16,), jnp.float32)
            ac2[sl] = jnp.zeros((16,), jnp.float32)
            return 0
        lax.fori_loop(0, OWN // 16, init_chunk, 0)

        def sweep(chunk_fn):
            def ch(ci, _):
                sl = pl.ds(ci * 16, 16)
                sv = Lsrc[sl]
                dv = Ldst[sl]
                valid = (ci * 16 + lane) < count_eff
                chunk_fn(sv, dv, valid)
                return 0
            lax.fori_loop(0, nch_list, ch, 0)

            @pl.when(count_true > CAP)
            def _():
                def rblk(b, cv):
                    pltpu.sync_copy(src_hbm.at[pl.ds(b * EB, EB)], eb_src)
                    pltpu.sync_copy(dst_hbm.at[pl.ds(b * EB, EB)], eb_dst)

                    def rch(ci, cv2):
                        sl = pl.ds(ci * 16, 16)
                        sv = eb_src[sl]
                        dv = eb_dst[sl]
                        mk = (dv // OWN) == wid
                        cs = plsc.cumsum(mk.astype(jnp.int32))
                        pos = cv2 + cs - 1
                        chunk_fn(sv, dv, mk & (pos >= CAP))
                        return cv2 + plsc.all_reduce_population_count(mk)
                    return lax.fori_loop(0, EB // 16, rch, cv)
                lax.fori_loop(0, EE // EB, rblk,
                              jnp.zeros((16,), jnp.int32) + OWN)

        def logit_of(sv, dv):
            lg = plsc.load_gather(ssrc, [sv]) + plsc.load_gather(sdst, [dv])
            return jnp.maximum(lg, 0.2 * lg)

        def s1(sv, dv, valid):
            logit = logit_of(sv, dv)

            def cond(p):
                return jnp.max(p) > 0

            def body(p):
                cur = plsc.load_gather(m, [dv])
                need = (p > 0) & (logit > cur)
                plsc.store_scatter(m, [dv], logit, mask=need)
                cur2 = plsc.load_gather(m, [dv])
                return jnp.where(need & (cur2 < logit), 1, 0).astype(jnp.int32)

            lax.while_loop(cond, body, valid.astype(jnp.int32))
        sweep(s1)

        def s2(sv, dv, valid):
            logit = logit_of(sv, dv)
            mg = plsc.load_gather(m, [dv])
            ex = jnp.exp(logit - mg)
            ex = jnp.where(valid, ex, 0.0)
            plsc.addupdate_scatter(den, [dv], ex, mask=valid)
            g0 = plsc.load_gather(hw0, [sv])
            g1 = plsc.load_gather(hw1, [sv])
            g2 = plsc.load_gather(hw2, [sv])
            plsc.addupdate_scatter(ac0, [dv], ex * g0, mask=valid)
            plsc.addupdate_scatter(ac1, [dv], ex * g1, mask=valid)
            plsc.addupdate_scatter(ac2, [dv], ex * g2, mask=valid)
        sweep(s2)

        def upd_chunk(i, _):
            sl = pl.ds(base + i * 16, 16)
            d_ = den[sl] + 1e-16
            n0 = ac0[sl] / d_ + bv[0]
            n1 = ac1[sl] / d_ + bv[1]
            n2 = ac2[sl] / d_ + bv[2]
            last = l >= NL - 1
            h0[sl] = jnp.where(last, n0, jnp.maximum(n0, 0.0))
            h1[sl] = jnp.where(last, n1, jnp.maximum(n1, 0.0))
            h2[sl] = jnp.where(last, n2, jnp.maximum(n2, 0.0))
            return 0
        lax.fori_loop(0, OWN // 16, upd_chunk, 0)

        pltpu.sync_copy(h0.at[pl.ds(base, OWN)], hT_out.at[pl.ds(base, OWN)])
        pltpu.sync_copy(h1.at[pl.ds(base, OWN)], hT_out.at[pl.ds(NN + base, OWN)])
        pltpu.sync_copy(h2.at[pl.ds(base, OWN)], hT_out.at[pl.ds(2 * NN + base, OWN)])
        plsc.subcore_barrier()
        pltpu.sync_copy(hT_out.at[pl.ds(0, NN)], h0)
        pltpu.sync_copy(hT_out.at[pl.ds(NN, NN)], h1)
        pltpu.sync_copy(hT_out.at[pl.ds(2 * NN, NN)], h2)
        plsc.subcore_barrier()
        return carry

    lax.fori_loop(0, NL, layer_body, 0)


def _gat_sc(src_e, dst_e, xT, wvec):
    mesh = plsc.VectorSubcoreMesh(core_axis_name="c", subcore_axis_name="s",
                                  num_cores=1, num_subcores=NSUB)
    fvec = lambda n: pltpu.VMEM((n,), jnp.float32)
    return pl.kernel(
        _gat_sc_body,
        out_type=jax.ShapeDtypeStruct((DD * NN,), jnp.float32),
        mesh=mesh,
        scratch_types=[
            pltpu.VMEM((EB,), jnp.int32),
            pltpu.VMEM((EB,), jnp.int32),
            pltpu.VMEM((CAP,), jnp.int32),
            pltpu.VMEM((CAP,), jnp.int32),
            fvec(192),
            fvec(NN), fvec(NN), fvec(NN),
            fvec(NN), fvec(NN), fvec(NN),
            fvec(NN), fvec(NN),
            fvec(NN), fvec(NN),
            fvec(NN), fvec(NN), fvec(NN),
        ],
        compiler_params=pltpu.CompilerParams(needs_layout_passes=False),
    )(src_e, dst_e, xT, wvec)


def _attn_mlp_body(h_ref, hT_ref, mask_ref,
                   Wq_ref, bq_ref, WkT_ref, bk_ref, WvT_ref, bv_ref,
                   Wo_ref, bo_ref, W1_ref, b1_ref, W2_ref, b2_ref,
                   W3_ref, b3_ref, out_ref):
    h_t = h_ref[...]
    hT = hT_ref[...]
    Q = jnp.dot(h_t, Wq_ref[...], preferred_element_type=jnp.float32) + bq_ref[...][None, :]
    kT = jnp.dot(WkT_ref[...], hT, preferred_element_type=jnp.float32) + bk_ref[...][:, None]
    vT = jnp.dot(WvT_ref[...], hT, preferred_element_type=jnp.float32) + bv_ref[...][:, None]
    cols = []
    for hh in range(3):
        qcol = Q[:, hh:hh + 1]
        krow = kT[hh:hh + 1, :]
        vrow = vT[hh:hh + 1, :]
        kmax = jnp.max(krow, axis=1, keepdims=True)
        kmin = jnp.min(krow, axis=1, keepdims=True)
        mrow = jnp.where(qcol >= 0.0, qcol * kmax, qcol * kmin)
        ex = jnp.exp(qcol * krow - mrow)
        dn = jnp.sum(ex, axis=1, keepdims=True)
        nm = jnp.sum(ex * vrow, axis=1, keepdims=True)
        cols.append(nm / dn)
    mo = jnp.concatenate(cols, axis=1)
    mo = jnp.dot(mo, Wo_ref[...], preferred_element_type=jnp.float32) + bo_ref[...][None, :]
    hm = jnp.maximum(jnp.dot(mo, W1_ref[...], preferred_element_type=jnp.float32) + b1_ref[...][None, :], 0.0)
    hm = jnp.maximum(jnp.dot(hm, W2_ref[...], preferred_element_type=jnp.float32) + b2_ref[...][None, :], 0.0)
    logits = jnp.dot(hm, W3_ref[...], preferred_element_type=jnp.float32) + b3_ref[...][None, :]
    out_ref[...] = jnp.where(mask_ref[...] == 1, logits, 0.0)


def _attn_mlp(h, hT, mask2d, Wq, bq, WkT, bk, WvT, bv, Wo, bo, W13, b1, W2, b2, W3, b3):
    full = lambda shape: pl.BlockSpec(shape, lambda i: tuple(0 for _ in shape))
    grid = NN // QT
    return pl.pallas_call(
        _attn_mlp_body,
        grid=(grid,),
        in_specs=[
            pl.BlockSpec((QT, DD), lambda i: (i, 0)),
            full((DD, NN)),
            pl.BlockSpec((QT, 1), lambda i: (i, 0)),
            full((DD, DD)), full((DD,)), full((DD, DD)), full((DD,)),
            full((DD, DD)), full((DD,)), full((DD, DD)), full((DD,)),
            full((DD, 16)), full((16,)), full((16, 32)), full((32,)),
            full((32, 1)), full((1,)),
        ],
        out_specs=pl.BlockSpec((QT, 1), lambda i: (i, 0)),
        out_shape=jax.ShapeDtypeStruct((NN, 1), jnp.float32),
    )(h, hT, mask2d, Wq, bq, WkT, bk, WvT, bv, Wo, bo, W13, b1, W2, b2, W3, b3)


def kernel(x, edge_index, mask, gat_W, gat_b, gat_a_src, gat_a_dst, Wq, bq, Wk, bk, Wv, bv, Wo, bo, W1, b1, W2, b2, W3, b3):
    wvec = jnp.concatenate([
        gat_W.reshape(-1), gat_a_src.reshape(-1), gat_a_dst.reshape(-1),
        gat_b.reshape(-1), jnp.zeros((12,), jnp.float32)])

    hT = _gat_sc(edge_index[0], edge_index[1], x.T.reshape(-1), wvec).reshape(DD, NN)
    h = hT.T

    logits = _attn_mlp(h, hT, mask[:, None].astype(jnp.int32),
                       Wq, bq, Wk.T, bk, Wv.T, bv, Wo, bo,
                       W1[:3, :], b1, W2, b2, W3, b3)
    return logits[:, 0]


# speedup vs baseline: 67.3243x; 1.3018x over previous
import functools

import jax
import jax.numpy as jnp
from jax import lax
from jax.experimental import pallas as pl
from jax.experimental.pallas import tpu as pltpu
from jax.experimental.pallas import tpu_sc as plsc

NN = 4096
DD = 3
NL = 10
QT = 256
EB = 8192
EE = 65536
CAP = 8192
NSUB = 16
OWN = NN // NSUB
NEG = float("-inf")


def _gat_sc_body(src_hbm, dst_hbm, xT_hbm, wv_hbm, hT_out,
                 eb_src, eb_dst, Lsrc, Ldst, wbuf,
                 h0, h1, h2, hw0, hw1, hw2, ssrc, sdst, m, den,
                 ac0, ac1, ac2):
    wid = lax.axis_index("s")
    base = wid * OWN
    lane = lax.iota(jnp.int32, 16)

    pltpu.sync_copy(wv_hbm, wbuf)
    pltpu.sync_copy(xT_hbm.at[pl.ds(0, NN)], h0)
    pltpu.sync_copy(xT_hbm.at[pl.ds(NN, NN)], h1)
    pltpu.sync_copy(xT_hbm.at[pl.ds(2 * NN, NN)], h2)

    def zero_chunk(i, _):
        sl = pl.ds(i * 16, 16)
        z = jnp.zeros((16,), jnp.int32)
        Lsrc[sl] = z
        Ldst[sl] = z
        return 0
    lax.fori_loop(0, CAP // 16, zero_chunk, 0)

    def loop_chunk(i, _):
        sl = pl.ds(i * 16, 16)
        ids = base + i * 16 + lane
        Lsrc[sl] = ids
        Ldst[sl] = ids
        return 0
    lax.fori_loop(0, OWN // 16, loop_chunk, 0)

    def scan_blk(b, countv):
        pltpu.sync_copy(src_hbm.at[pl.ds(b * EB, EB)], eb_src)
        pltpu.sync_copy(dst_hbm.at[pl.ds(b * EB, EB)], eb_dst)

        @plsc.parallel_loop(0, EB // 16, unroll=4, carry=countv)
        def ch(ci, cv):
            sl = pl.ds(ci * 16, 16)
            sv = eb_src[sl]
            dv = eb_dst[sl]
            mk = (dv // OWN) == wid
            cs = plsc.cumsum(mk.astype(jnp.int32))
            pos = cv + cs - 1
            ok = mk & (pos < CAP)
            plsc.store_scatter(Lsrc, [pos], sv, mask=ok)
            plsc.store_scatter(Ldst, [pos], dv, mask=ok)
            return cv + plsc.all_reduce_population_count(mk)
        return ch

    countv = lax.fori_loop(0, EE // EB, scan_blk,
                           jnp.zeros((16,), jnp.int32) + OWN)
    count_true = jnp.max(countv)
    count_eff = jnp.minimum(count_true, CAP)
    nch_list = (count_eff + 15) // 16

    def wb(i):
        return plsc.load_gather(wbuf, [jnp.zeros((16,), jnp.int32) + i])

    def layer_body(l, carry):
        w = [wb(9 * l + t) for t in range(9)]
        asv = [wb(90 + 3 * l + t) for t in range(3)]
        adv = [wb(120 + 3 * l + t) for t in range(3)]
        bv = [wb(150 + 3 * l + t) for t in range(3)]

        def dense_chunk(i, _=None):
            sl = pl.ds(i * 16, 16)
            a, b_, c = h0[sl], h1[sl], h2[sl]
            t0 = a * w[0] + b_ * w[3] + c * w[6]
            t1 = a * w[1] + b_ * w[4] + c * w[7]
            t2 = a * w[2] + b_ * w[5] + c * w[8]
            hw0[sl] = t0
            hw1[sl] = t1
            hw2[sl] = t2
            ssrc[sl] = t0 * asv[0] + t1 * asv[1] + t2 * asv[2]
            sdst[sl] = t0 * adv[0] + t1 * adv[1] + t2 * adv[2]
        plsc.parallel_loop(0, NN // 16, unroll=4)(dense_chunk)

        def init_chunk(i, _=None):
            sl = pl.ds(base + i * 16, 16)
            m[sl] = jnp.full((16,), NEG, jnp.float32)
            den[sl] = jnp.zeros((16,), jnp.float32)
            ac0[sl] = jnp.zeros((16,), jnp.float32)
            ac1[sl] = jnp.zeros((16,), jnp.float32)
            ac2[sl] = jnp.zeros((16,), jnp.float32)
        plsc.parallel_loop(0, OWN // 16, unroll=4)(init_chunk)

        def sweep(chunk_fn, par=False):
            def ch(ci, _=None):
                sl = pl.ds(ci * 16, 16)
                sv = Lsrc[sl]
                dv = Ldst[sl]
                valid = (ci * 16 + lane) < count_eff
                chunk_fn(sv, dv, valid)
                return 0
            if par:
                plsc.parallel_loop(0, nch_list, unroll=4)(ch)
            else:
                lax.fori_loop(0, nch_list, ch, 0)

            @pl.when(count_true > CAP)
            def _():
                def rblk(b, cv):
                    pltpu.sync_copy(src_hbm.at[pl.ds(b * EB, EB)], eb_src)
                    pltpu.sync_copy(dst_hbm.at[pl.ds(b * EB, EB)], eb_dst)

                    def rch(ci, cv2):
                        sl = pl.ds(ci * 16, 16)
                        sv = eb_src[sl]
                        dv = eb_dst[sl]
                        mk = (dv // OWN) == wid
                        cs = plsc.cumsum(mk.astype(jnp.int32))
                        pos = cv2 + cs - 1
                        chunk_fn(sv, dv, mk & (pos >= CAP))
                        return cv2 + plsc.all_reduce_population_count(mk)
                    return lax.fori_loop(0, EB // 16, rch, cv)
                lax.fori_loop(0, EE // EB, rblk,
                              jnp.zeros((16,), jnp.int32) + OWN)

        def logit_of(sv, dv):
            lg = plsc.load_gather(ssrc, [sv]) + plsc.load_gather(sdst, [dv])
            return jnp.maximum(lg, 0.2 * lg)

        def s1(sv, dv, valid):
            logit = logit_of(sv, dv)

            def cond(p):
                return jnp.max(p) > 0

            def body(p):
                cur = plsc.load_gather(m, [dv])
                need = (p > 0) & (logit > cur)
                plsc.store_scatter(m, [dv], logit, mask=need)
                cur2 = plsc.load_gather(m, [dv])
                return jnp.where(need & (cur2 < logit), 1, 0).astype(jnp.int32)

            lax.while_loop(cond, body, valid.astype(jnp.int32))
        sweep(s1)

        def s2(sv, dv, valid):
            logit = logit_of(sv, dv)
            mg = plsc.load_gather(m, [dv])
            ex = jnp.exp(logit - mg)
            ex = jnp.where(valid, ex, 0.0)
            plsc.addupdate_scatter(den, [dv], ex, mask=valid)
            g0 = plsc.load_gather(hw0, [sv])
            g1 = plsc.load_gather(hw1, [sv])
            g2 = plsc.load_gather(hw2, [sv])
            plsc.addupdate_scatter(ac0, [dv], ex * g0, mask=valid)
            plsc.addupdate_scatter(ac1, [dv], ex * g1, mask=valid)
            plsc.addupdate_scatter(ac2, [dv], ex * g2, mask=valid)
        sweep(s2, par=True)

        def upd_chunk(i, _=None):
            sl = pl.ds(base + i * 16, 16)
            d_ = den[sl] + 1e-16
            n0 = ac0[sl] / d_ + bv[0]
            n1 = ac1[sl] / d_ + bv[1]
            n2 = ac2[sl] / d_ + bv[2]
            last = l >= NL - 1
            h0[sl] = jnp.where(last, n0, jnp.maximum(n0, 0.0))
            h1[sl] = jnp.where(last, n1, jnp.maximum(n1, 0.0))
            h2[sl] = jnp.where(last, n2, jnp.maximum(n2, 0.0))
        plsc.parallel_loop(0, OWN // 16, unroll=4)(upd_chunk)

        pltpu.sync_copy(h0.at[pl.ds(base, OWN)], hT_out.at[pl.ds(base, OWN)])
        pltpu.sync_copy(h1.at[pl.ds(base, OWN)], hT_out.at[pl.ds(NN + base, OWN)])
        pltpu.sync_copy(h2.at[pl.ds(base, OWN)], hT_out.at[pl.ds(2 * NN + base, OWN)])
        plsc.subcore_barrier()
        pltpu.sync_copy(hT_out.at[pl.ds(0, NN)], h0)
        pltpu.sync_copy(hT_out.at[pl.ds(NN, NN)], h1)
        pltpu.sync_copy(hT_out.at[pl.ds(2 * NN, NN)], h2)
        plsc.subcore_barrier()
        return carry

    lax.fori_loop(0, NL, layer_body, 0)


def _gat_sc(src_e, dst_e, xT, wvec):
    mesh = plsc.VectorSubcoreMesh(core_axis_name="c", subcore_axis_name="s",
                                  num_cores=1, num_subcores=NSUB)
    fvec = lambda n: pltpu.VMEM((n,), jnp.float32)
    return pl.kernel(
        _gat_sc_body,
        out_type=jax.ShapeDtypeStruct((DD * NN,), jnp.float32),
        mesh=mesh,
        scratch_types=[
            pltpu.VMEM((EB,), jnp.int32),
            pltpu.VMEM((EB,), jnp.int32),
            pltpu.VMEM((CAP,), jnp.int32),
            pltpu.VMEM((CAP,), jnp.int32),
            fvec(192),
            fvec(NN), fvec(NN), fvec(NN),
            fvec(NN), fvec(NN), fvec(NN),
            fvec(NN), fvec(NN),
            fvec(NN), fvec(NN),
            fvec(NN), fvec(NN), fvec(NN),
        ],
        compiler_params=pltpu.CompilerParams(needs_layout_passes=False),
    )(src_e, dst_e, xT, wvec)


def _attn_mlp_body(h_ref, hT_ref, mask_ref,
                   Wq_ref, bq_ref, WkT_ref, bk_ref, WvT_ref, bv_ref,
                   Wo_ref, bo_ref, W1_ref, b1_ref, W2_ref, b2_ref,
                   W3_ref, b3_ref, out_ref):
    h_t = h_ref[...]
    hT = hT_ref[...]
    Q = jnp.dot(h_t, Wq_ref[...], preferred_element_type=jnp.float32) + bq_ref[...][None, :]
    kT = jnp.dot(WkT_ref[...], hT, preferred_element_type=jnp.float32) + bk_ref[...][:, None]
    vT = jnp.dot(WvT_ref[...], hT, preferred_element_type=jnp.float32) + bv_ref[...][:, None]
    cols = []
    for hh in range(3):
        qcol = Q[:, hh:hh + 1]
        krow = kT[hh:hh + 1, :]
        vrow = vT[hh:hh + 1, :]
        kmax = jnp.max(krow, axis=1, keepdims=True)
        kmin = jnp.min(krow, axis=1, keepdims=True)
        mrow = jnp.where(qcol >= 0.0, qcol * kmax, qcol * kmin)
        ex = jnp.exp(qcol * krow - mrow)
        dn = jnp.sum(ex, axis=1, keepdims=True)
        nm = jnp.sum(ex * vrow, axis=1, keepdims=True)
        cols.append(nm / dn)
    mo = jnp.concatenate(cols, axis=1)
    mo = jnp.dot(mo, Wo_ref[...], preferred_element_type=jnp.float32) + bo_ref[...][None, :]
    hm = jnp.maximum(jnp.dot(mo, W1_ref[...], preferred_element_type=jnp.float32) + b1_ref[...][None, :], 0.0)
    hm = jnp.maximum(jnp.dot(hm, W2_ref[...], preferred_element_type=jnp.float32) + b2_ref[...][None, :], 0.0)
    logits = jnp.dot(hm, W3_ref[...], preferred_element_type=jnp.float32) + b3_ref[...][None, :]
    out_ref[...] = jnp.where(mask_ref[...] == 1, logits, 0.0)


def _attn_mlp(h, hT, mask2d, Wq, bq, WkT, bk, WvT, bv, Wo, bo, W13, b1, W2, b2, W3, b3):
    full = lambda shape: pl.BlockSpec(shape, lambda i: tuple(0 for _ in shape))
    grid = NN // QT
    return pl.pallas_call(
        _attn_mlp_body,
        grid=(grid,),
        in_specs=[
            pl.BlockSpec((QT, DD), lambda i: (i, 0)),
            full((DD, NN)),
            pl.BlockSpec((QT, 1), lambda i: (i, 0)),
            full((DD, DD)), full((DD,)), full((DD, DD)), full((DD,)),
            full((DD, DD)), full((DD,)), full((DD, DD)), full((DD,)),
            full((DD, 16)), full((16,)), full((16, 32)), full((32,)),
            full((32, 1)), full((1,)),
        ],
        out_specs=pl.BlockSpec((QT, 1), lambda i: (i, 0)),
        out_shape=jax.ShapeDtypeStruct((NN, 1), jnp.float32),
    )(h, hT, mask2d, Wq, bq, WkT, bk, WvT, bv, Wo, bo, W13, b1, W2, b2, W3, b3)


def kernel(x, edge_index, mask, gat_W, gat_b, gat_a_src, gat_a_dst, Wq, bq, Wk, bk, Wv, bv, Wo, bo, W1, b1, W2, b2, W3, b3):
    wvec = jnp.concatenate([
        gat_W.reshape(-1), gat_a_src.reshape(-1), gat_a_dst.reshape(-1),
        gat_b.reshape(-1), jnp.zeros((12,), jnp.float32)])

    hT = _gat_sc(edge_index[0], edge_index[1], x.T.reshape(-1), wvec).reshape(DD, NN)
    h = hT.T

    logits = _attn_mlp(h, hT, mask[:, None].astype(jnp.int32),
                       Wq, bq, Wk.T, bk, Wv.T, bv, Wo, bo,
                       W1[:3, :], b1, W2, b2, W3, b3)
    return logits[:, 0]
